# baseline (speedup 1.0000x reference)
import jax
import jax.numpy as jnp
from jax import lax
from jax.experimental import pallas as pl
from jax.experimental.pallas import tpu as pltpu
from jax.experimental.pallas import tpu_sc as plsc

B, S, H = 4, 2048, 768
NT = B * S
LANES = 16
NG = H // LANES
EPS = 1e-12

NC, NS = 2, 16
NW = NC * NS
TPW = NT // NW
T = 32
NCHUNK = TPW // T


def _rsqrt(v):
    i = lax.bitcast_convert_type(v, jnp.int32)
    i = jnp.int32(0x5F3759DF) - (i >> 1)
    y = lax.bitcast_convert_type(i, jnp.float32)
    for _ in range(4):
        y = y * (1.5 - 0.5 * v * y * y)
    return y


def _body(ids_hbm, tt_hbm, tok_hbm, pos_hbm, type_hbm, gamma_hbm, beta_hbm,
          out_hbm, ids_v, tt_v, tok_buf, pos_buf, type_v, gamma_v, beta_v,
          gsem):
    wid = lax.axis_index("s") * NC + lax.axis_index("c")
    base = wid * TPW
    pos0 = lax.rem(base, S)

    pltpu.sync_copy(ids_hbm.at[pl.ds(base, TPW)], ids_v)
    pltpu.sync_copy(tt_hbm.at[pl.ds(base, TPW)], tt_v)
    pltpu.sync_copy(type_hbm, type_v)
    pltpu.sync_copy(gamma_hbm, gamma_v)
    pltpu.sync_copy(beta_hbm, beta_v)

    def diff_body(j, _):
        sl = pl.ds(j * LANES, LANES)
        type_v[1, sl] = type_v[1, sl] - type_v[0, sl]
        return 0

    lax.fori_loop(0, NG, diff_body, 0)

    def chunk_body(g, _):
        pltpu.async_copy(
            tok_hbm.at[ids_v.at[pl.ds(g * T, T)]], tok_buf, gsem).wait()
        pltpu.sync_copy(pos_hbm.at[pl.ds(pos0 + g * T, T)], pos_buf)

        def grp_body(q, _):
            c16 = tt_v[pl.ds(g * T + q * LANES, LANES)].astype(jnp.float32)
            for u in range(LANES):
                t = q * LANES + u
                c_f = c16[u]

                def p1(j, carry):
                    s, s2 = carry
                    sl = pl.ds(j * LANES, LANES)
                    x = (tok_buf[t, sl] + pos_buf[t, sl]
                         + type_v[0, sl] + c_f * type_v[1, sl])
                    tok_buf[t, sl] = x
                    return (s + x, s2 + x * x)

                zero = jnp.zeros((LANES,), jnp.float32)
                s, s2 = lax.fori_loop(0, NG, p1, (zero, zero))
                mean = jnp.sum(s) * (1.0 / H)
                var = jnp.sum(s2) * (1.0 / H) - mean * mean
                rstd = _rsqrt(var + EPS)

                def p2(j, _):
                    sl = pl.ds(j * LANES, LANES)
                    x = tok_buf[t, sl]
                    tok_buf[t, sl] = ((x - mean) * rstd * gamma_v[sl]
                                      + beta_v[sl])
                    return 0

                lax.fori_loop(0, NG, p2, 0)
            return 0

        lax.fori_loop(0, T // LANES, grp_body, 0)
        pltpu.sync_copy(tok_buf, out_hbm.at[pl.ds(base + g * T, T)])
        return 0

    lax.fori_loop(0, NCHUNK, chunk_body, 0)


def kernel(input_ids, token_type_ids, token_embedding, position_embedding,
           token_type_embedding, ln_gamma, ln_beta):
    ids = input_ids.reshape(NT)
    tts = token_type_ids.reshape(NT)
    mesh = plsc.VectorSubcoreMesh(core_axis_name="c", subcore_axis_name="s")
    run = pl.kernel(
        _body,
        out_type=jax.ShapeDtypeStruct((NT, H), jnp.float32),
        mesh=mesh,
        compiler_params=pltpu.CompilerParams(needs_layout_passes=False),
        scratch_types=[
            pltpu.VMEM((TPW,), jnp.int32),
            pltpu.VMEM((TPW,), jnp.int32),
            pltpu.VMEM((T, H), jnp.float32),
            pltpu.VMEM((T, H), jnp.float32),
            pltpu.VMEM((2, H), jnp.float32),
            pltpu.VMEM((H,), jnp.float32),
            pltpu.VMEM((H,), jnp.float32),
            pltpu.SemaphoreType.DMA,
        ],
    )
    out = run(ids, tts, token_embedding, position_embedding,
              token_type_embedding, ln_gamma, ln_beta)
    return out.reshape(B, S, H)


# speedup vs baseline: 1.0368x; 1.0368x over previous
import jax
import jax.numpy as jnp
from jax import lax
from jax.experimental import pallas as pl
from jax.experimental.pallas import tpu as pltpu
from jax.experimental.pallas import tpu_sc as plsc

B, S, H = 4, 2048, 768
NT = B * S
LANES = 16
NG = H // LANES
EPS = 1e-12

NC, NS = 2, 16
NW = NC * NS
TPW = NT // NW
T = 32
NCHUNK = TPW // T
UNROLL = 8


def _rsqrt_v(v):
    i = lax.bitcast_convert_type(v, jnp.int32)
    i = jnp.int32(0x5F3759DF) - (i >> 1)
    y = lax.bitcast_convert_type(i, jnp.float32)
    for _ in range(4):
        y = y * (1.5 - 0.5 * v * y * y)
    return y


def _body(ids_hbm, tt_hbm, tok_hbm, pos_hbm, type_hbm, gamma_hbm, beta_hbm,
          out_hbm, ids_v, tt_v, tok_buf, pos_buf, type_v, gamma_v, beta_v,
          sbuf, gsem, psem, osem):
    wid = lax.axis_index("s") * NC + lax.axis_index("c")
    base = wid * TPW
    pos0 = lax.rem(base, S)

    pltpu.sync_copy(ids_hbm.at[pl.ds(base, TPW)], ids_v)
    pltpu.sync_copy(tt_hbm.at[pl.ds(base, TPW)], tt_v)
    pltpu.sync_copy(type_hbm, type_v)
    pltpu.sync_copy(gamma_hbm, gamma_v)
    pltpu.sync_copy(beta_hbm, beta_v)

    def start_in(g, slot):
        pltpu.async_copy(tok_hbm.at[ids_v.at[pl.ds(g * T, T)]],
                         tok_buf.at[slot], gsem.at[slot])
        pltpu.async_copy(pos_hbm.at[pl.ds(pos0 + g * T, T)],
                         pos_buf.at[slot], psem.at[slot])

    def wait_in(slot):
        pltpu.make_async_copy(tok_hbm.at[pl.ds(0, T)], tok_buf.at[slot],
                              gsem.at[slot]).wait()
        pltpu.make_async_copy(pos_hbm.at[pl.ds(0, T)], pos_buf.at[slot],
                              psem.at[slot]).wait()

    def start_out(g, slot):
        pltpu.async_copy(tok_buf.at[slot], out_hbm.at[pl.ds(base + g * T, T)],
                         osem.at[slot])

    def wait_out(slot):
        pltpu.make_async_copy(tok_buf.at[slot], out_hbm.at[pl.ds(0, T)],
                              osem.at[slot]).wait()

    start_in(0, 0)

    def chunk_body(g, _):
        p = g & 1

        @pl.when(jnp.logical_and(g >= 1, g < NCHUNK - 1))
        def _():
            wait_out(1 - p)

        @pl.when(g < NCHUNK - 1)
        def _():
            start_in(g + 1, 1 - p)

        wait_in(p)

        def grp_body(q, _):
            c16 = tt_v[pl.ds(g * T + q * LANES, LANES)]
            for u in range(LANES):
                t = q * LANES + u
                c_i = c16[u]

                def p1(j, carry):
                    s, s2 = carry
                    sl = pl.ds(j * LANES, LANES)
                    x = (tok_buf[p, t, sl] + pos_buf[p, t, sl]
                         + type_v[c_i, sl])
                    tok_buf[p, t, sl] = x
                    return (s + x, s2 + x * x)

                zero = jnp.zeros((LANES,), jnp.float32)
                s, s2 = lax.fori_loop(0, NG, p1, (zero, zero),
                                      unroll=UNROLL)
                sbuf[0, u, :] = s
                sbuf[1, u, :] = s2

            lane_iota = lax.iota(jnp.int32, LANES)
            sum_v = jnp.zeros((LANES,), jnp.float32)
            sq_v = jnp.zeros((LANES,), jnp.float32)
            for k in range(LANES):
                zi = jnp.full((LANES,), 0, jnp.int32)
                oi = jnp.full((LANES,), 1, jnp.int32)
                ki = jnp.full((LANES,), k, jnp.int32)
                sum_v = sum_v + plsc.load_gather(sbuf, [zi, lane_iota, ki])
                sq_v = sq_v + plsc.load_gather(sbuf, [oi, lane_iota, ki])
            mean_v = sum_v * (1.0 / H)
            var_v = sq_v * (1.0 / H) - mean_v * mean_v
            rstd_v = _rsqrt_v(var_v + EPS)
            mr_v = mean_v * rstd_v

            for u in range(LANES):
                t = q * LANES + u
                r = rstd_v[u]
                mr = mr_v[u]

                def p2(j, _):
                    sl = pl.ds(j * LANES, LANES)
                    x = tok_buf[p, t, sl]
                    tok_buf[p, t, sl] = ((x * r - mr) * gamma_v[sl]
                                         + beta_v[sl])
                    return 0

                lax.fori_loop(0, NG, p2, 0, unroll=UNROLL)
            return 0

        lax.fori_loop(0, T // LANES, grp_body, 0)
        start_out(g, p)
        return 0

    lax.fori_loop(0, NCHUNK, chunk_body, 0)
    wait_out(0)
    wait_out(1)


def kernel(input_ids, token_type_ids, token_embedding, position_embedding,
           token_type_embedding, ln_gamma, ln_beta):
    ids = input_ids.reshape(NT)
    tts = token_type_ids.reshape(NT)
    mesh = plsc.VectorSubcoreMesh(core_axis_name="c", subcore_axis_name="s")
    run = pl.kernel(
        _body,
        out_type=jax.ShapeDtypeStruct((NT, H), jnp.float32),
        mesh=mesh,
        compiler_params=pltpu.CompilerParams(needs_layout_passes=False),
        scratch_types=[
            pltpu.VMEM((TPW,), jnp.int32),
            pltpu.VMEM((TPW,), jnp.int32),
            pltpu.VMEM((2, T, H), jnp.float32),
            pltpu.VMEM((2, T, H), jnp.float32),
            pltpu.VMEM((2, H), jnp.float32),
            pltpu.VMEM((H,), jnp.float32),
            pltpu.VMEM((H,), jnp.float32),
            pltpu.VMEM((2, LANES, LANES), jnp.float32),
            pltpu.SemaphoreType.DMA((2,)),
            pltpu.SemaphoreType.DMA((2,)),
            pltpu.SemaphoreType.DMA((2,)),
        ],
    )
    out = run(ids, tts, token_embedding, position_embedding,
              token_type_embedding, ln_gamma, ln_beta)
    return out.reshape(B, S, H)
